# Initial kernel scaffold; baseline (speedup 1.0000x reference)
import jax
import jax.numpy as jnp
from jax.experimental import pallas as pl


def kernel(pos, edge_index, W0, as0, ad0, b0, W1, as1, ad1, b1, W2, as2, ad2, b2):
    raise NotImplementedError("write your pallas kernel here")


# speedup vs baseline: 36.4741x; 36.4741x over previous
import functools

import jax
import jax.numpy as jnp
from jax import lax
from jax.experimental import pallas as pl
from jax.experimental.pallas import tpu as pltpu
from jax.experimental.pallas import tpu_sc as plsc

F32 = jnp.float32

N = 50000
NPAD = 51200
E = 850000
EPAD = 851968
ROWS = EPAD // 128
NC, NS = 2, 16
ROWS_PER_SUB = ROWS // NS
SUPER = ROWS_PER_SUB // 16
ROWS_PER_SUB0 = ROWS // (NC * NS)
SUPER0 = ROWS_PER_SUB0 // 16
NSLICE = NPAD // NS
BLK = 2048
GRID = NPAD // BLK
NEG = -3.0e38
EPS = 1e-16

_MESH = plsc.VectorSubcoreMesh(core_axis_name="c", subcore_axis_name="s",
                               num_cores=NC, num_subcores=NS)
_SC_PARAMS = pltpu.CompilerParams(use_tc_tiling_on_sc=False)


def _tc_a_body(pos4_ref, w0_ref, as0_ref, ad0_ref,
               xa0_ref, asrc_ref, adst_ref, m_ref, msum_ref):
    i = pl.program_id(0)

    @pl.when(i == 0)
    def _():
        msum_ref[0] = NEG
        msum_ref[1] = NEG

    p4 = pos4_ref[...]
    x3 = p4[:, 0:3]
    was = w0_ref[...] @ as0_ref[...]
    wad = w0_ref[...] @ ad0_ref[...]
    asrc = x3 @ was
    adst = x3 @ wad
    xa0_ref[...] = jnp.concatenate(
        [p4, jnp.zeros((BLK, 12), F32)], axis=1)
    asrc_ref[...] = asrc
    adst_ref[...] = adst
    msum_ref[0] = jnp.maximum(msum_ref[0], jnp.max(asrc))
    msum_ref[1] = jnp.maximum(msum_ref[1], jnp.max(adst))

    @pl.when(i == GRID - 1)
    def _():
        z = msum_ref[0] + msum_ref[1]
        m = jnp.maximum(z, F32(0.2) * z)
        m_ref[...] = jnp.full((8, 128), m, F32)


_tc_a = pl.pallas_call(
    _tc_a_body,
    grid=(GRID,),
    in_specs=[
        pl.BlockSpec((BLK, 4), lambda i: (i, 0)),
        pl.BlockSpec((3, 64), lambda i: (0, 0)),
        pl.BlockSpec((64, 1), lambda i: (0, 0)),
        pl.BlockSpec((64, 1), lambda i: (0, 0)),
    ],
    out_specs=[
        pl.BlockSpec((BLK, 16), lambda i: (i, 0)),
        pl.BlockSpec((BLK, 1), lambda i: (i, 0)),
        pl.BlockSpec((BLK, 1), lambda i: (i, 0)),
        pl.BlockSpec((8, 128), lambda i: (0, 0)),
    ],
    out_shape=[
        jax.ShapeDtypeStruct((NPAD, 16), F32),
        jax.ShapeDtypeStruct((NPAD, 1), F32),
        jax.ShapeDtypeStruct((NPAD, 1), F32),
        jax.ShapeDtypeStruct((8, 128), F32),
    ],
    scratch_shapes=[pltpu.SMEM((2,), F32)],
)


def _tc_b_body(aga_ref, agb_ref, w0_ref, b0_ref, w1_ref, as1_ref, ad1_ref,
               xlo_ref, xhi_ref, asrc_ref, adst_ref, m_ref, msum_ref):
    i = pl.program_id(0)

    @pl.when(i == 0)
    def _():
        msum_ref[0] = NEG
        msum_ref[1] = NEG

    agg = aga_ref[...] + agb_ref[...]
    den = agg[:, 3:4] + EPS
    xn = agg[:, 0:3] / den
    x1 = jnp.maximum(xn @ w0_ref[...] + b0_ref[...], 0.0)
    xlo_ref[...] = x1[:, 0:32]
    xhi_ref[...] = x1[:, 32:64]
    was = w1_ref[...] @ as1_ref[...]
    wad = w1_ref[...] @ ad1_ref[...]
    asrc = x1 @ was
    adst = x1 @ wad
    asrc_ref[...] = asrc
    adst_ref[...] = adst
    msum_ref[0] = jnp.maximum(msum_ref[0], jnp.max(asrc))
    msum_ref[1] = jnp.maximum(msum_ref[1], jnp.max(adst))

    @pl.when(i == GRID - 1)
    def _():
        z = msum_ref[0] + msum_ref[1]
        m = jnp.maximum(z, F32(0.2) * z)
        m_ref[...] = jnp.full((8, 128), m, F32)


_tc_b = pl.pallas_call(
    _tc_b_body,
    grid=(GRID,),
    in_specs=[
        pl.BlockSpec((BLK, 16), lambda i: (i, 0)),
        pl.BlockSpec((BLK, 16), lambda i: (i, 0)),
        pl.BlockSpec((3, 64), lambda i: (0, 0)),
        pl.BlockSpec((1, 64), lambda i: (0, 0)),
        pl.BlockSpec((64, 64), lambda i: (0, 0)),
        pl.BlockSpec((64, 1), lambda i: (0, 0)),
        pl.BlockSpec((64, 1), lambda i: (0, 0)),
    ],
    out_specs=[
        pl.BlockSpec((BLK, 32), lambda i: (i, 0)),
        pl.BlockSpec((BLK, 32), lambda i: (i, 0)),
        pl.BlockSpec((BLK, 1), lambda i: (i, 0)),
        pl.BlockSpec((BLK, 1), lambda i: (i, 0)),
        pl.BlockSpec((8, 128), lambda i: (0, 0)),
    ],
    out_shape=[
        jax.ShapeDtypeStruct((NPAD, 32), F32),
        jax.ShapeDtypeStruct((NPAD, 32), F32),
        jax.ShapeDtypeStruct((NPAD, 1), F32),
        jax.ShapeDtypeStruct((NPAD, 1), F32),
        jax.ShapeDtypeStruct((8, 128), F32),
    ],
    scratch_shapes=[pltpu.SMEM((2,), F32)],
)


def _tc_c_body(aglo_ref, aghi_ref, den_ref, w1_ref, b1_ref, w2_ref,
               as2_ref, ad2_ref,
               xlo_ref, xhi_ref, asrc_ref, adst_ref, m_ref, msum_ref):
    i = pl.program_id(0)

    @pl.when(i == 0)
    def _():
        msum_ref[0] = NEG
        msum_ref[1] = NEG

    inv = 1.0 / (den_ref[...] + EPS)
    xn_lo = aglo_ref[...] * inv
    xn_hi = aghi_ref[...] * inv
    w1 = w1_ref[...]
    x2 = jnp.maximum(
        xn_lo @ w1[0:32, :] + xn_hi @ w1[32:64, :] + b1_ref[...], 0.0)
    xlo_ref[...] = x2[:, 0:32]
    xhi_ref[...] = x2[:, 32:64]
    was = w2_ref[...] @ as2_ref[...]
    wad = w2_ref[...] @ ad2_ref[...]
    asrc = x2 @ was
    adst = x2 @ wad
    asrc_ref[...] = asrc
    adst_ref[...] = adst
    msum_ref[0] = jnp.maximum(msum_ref[0], jnp.max(asrc))
    msum_ref[1] = jnp.maximum(msum_ref[1], jnp.max(adst))

    @pl.when(i == GRID - 1)
    def _():
        z = msum_ref[0] + msum_ref[1]
        m = jnp.maximum(z, F32(0.2) * z)
        m_ref[...] = jnp.full((8, 128), m, F32)


_tc_c = pl.pallas_call(
    _tc_c_body,
    grid=(GRID,),
    in_specs=[
        pl.BlockSpec((BLK, 32), lambda i: (i, 0)),
        pl.BlockSpec((BLK, 32), lambda i: (i, 0)),
        pl.BlockSpec((BLK, 1), lambda i: (i, 0)),
        pl.BlockSpec((64, 64), lambda i: (0, 0)),
        pl.BlockSpec((1, 64), lambda i: (0, 0)),
        pl.BlockSpec((64, 68), lambda i: (0, 0)),
        pl.BlockSpec((68, 1), lambda i: (0, 0)),
        pl.BlockSpec((68, 1), lambda i: (0, 0)),
    ],
    out_specs=[
        pl.BlockSpec((BLK, 32), lambda i: (i, 0)),
        pl.BlockSpec((BLK, 32), lambda i: (i, 0)),
        pl.BlockSpec((BLK, 1), lambda i: (i, 0)),
        pl.BlockSpec((BLK, 1), lambda i: (i, 0)),
        pl.BlockSpec((8, 128), lambda i: (0, 0)),
    ],
    out_shape=[
        jax.ShapeDtypeStruct((NPAD, 32), F32),
        jax.ShapeDtypeStruct((NPAD, 32), F32),
        jax.ShapeDtypeStruct((NPAD, 1), F32),
        jax.ShapeDtypeStruct((NPAD, 1), F32),
        jax.ShapeDtypeStruct((8, 128), F32),
    ],
    scratch_shapes=[pltpu.SMEM((2,), F32)],
)


def _tc_d_body(aglo_ref, aghi_ref, den_ref, w2_ref, b2_ref, out_ref):
    inv = 1.0 / (den_ref[...] + EPS)
    xn_lo = aglo_ref[...] * inv
    xn_hi = aghi_ref[...] * inv
    w2 = w2_ref[...]
    out_ref[...] = xn_lo @ w2[0:32, :] + xn_hi @ w2[32:64, :] + b2_ref[...]


_tc_d = pl.pallas_call(
    _tc_d_body,
    grid=(GRID,),
    in_specs=[
        pl.BlockSpec((BLK, 32), lambda i: (i, 0)),
        pl.BlockSpec((BLK, 32), lambda i: (i, 0)),
        pl.BlockSpec((BLK, 1), lambda i: (i, 0)),
        pl.BlockSpec((64, 68), lambda i: (0, 0)),
        pl.BlockSpec((1, 68), lambda i: (0, 0)),
    ],
    out_specs=[pl.BlockSpec((BLK, 68), lambda i: (i, 0))],
    out_shape=[jax.ShapeDtypeStruct((NPAD, 68), F32)],
)


def _zero_vmem(zr, zd):
    zero16 = jnp.zeros((16,), F32)

    def _zr_body(r, _):
        zr[r, pl.ds(0, 16)] = zero16
        zr[r, pl.ds(16, 16)] = zero16
        return 0

    lax.fori_loop(0, 128, _zr_body, 0)
    for k in range(8):
        zd[pl.ds(16 * k, 16)] = zero16


def _sc_l12_body(src_h, dst_h, asrc_h, adst_h, m_h, xlo_h, xhi_h,
                 out_lo, out_hi, den_o,
                 acc_sp, den_sp, idx_s, idx_d, av, dv, ev, rows, mv,
                 zr, zd, sem, sem2):
    c = lax.axis_index("c")
    s = lax.axis_index("s")

    _zero_vmem(zr, zd)
    base = s * NSLICE
    for k in range(NSLICE // 128):
        off = pl.multiple_of(base + k * 128, 128)
        pltpu.sync_copy(zr, acc_sp.at[pl.ds(off, 128), :])
        pltpu.sync_copy(zd, den_sp.at[pl.ds(off, 128)])
    plsc.subcore_barrier()

    pltpu.sync_copy(m_h, mv)
    mvec = mv[...]

    def _super(t, _):
        row0 = s * ROWS_PER_SUB + t * 16
        pltpu.sync_copy(src_h.at[pl.ds(row0, 16), :], idx_s)
        pltpu.sync_copy(dst_h.at[pl.ds(row0, 16), :], idx_d)
        descs = []
        for j in range(16):
            descs.append(
                pltpu.async_copy(asrc_h.at[idx_s.at[j]],
                                 av.at[pl.ds(j * 128, 128)], sem))
            descs.append(
                pltpu.async_copy(adst_h.at[idx_d.at[j]],
                                 dv.at[pl.ds(j * 128, 128)], sem))
        for dsc in descs:
            dsc.wait()

        def _e_body(g, _):
            z = av[pl.ds(g * 16, 16)] + dv[pl.ds(g * 16, 16)]
            l = jnp.maximum(z, F32(0.2) * z)
            ev[pl.ds(g * 16, 16)] = jnp.exp(l - mvec)
            return 0

        lax.fori_loop(0, 128, _e_body, 0)

        for j in range(16):
            @pl.when(c == 0)
            def _():
                pltpu.async_copy(xlo_h.at[idx_s.at[j]], rows, sem2).wait()

            @pl.when(c == 1)
            def _():
                pltpu.async_copy(xhi_h.at[idx_s.at[j]], rows, sem2).wait()

            eb = j * 128

            def _sc_body(g, _):
                e16 = ev[pl.ds(eb + g * 16, 16)]
                r0 = g * 16
                for k in range(16):
                    scl = lax.broadcast(e16[k], (16,))
                    rows[r0 + k, pl.ds(0, 16)] = (
                        rows[r0 + k, pl.ds(0, 16)] * scl)
                    rows[r0 + k, pl.ds(16, 16)] = (
                        rows[r0 + k, pl.ds(16, 16)] * scl)
                return 0

            lax.fori_loop(0, 8, _sc_body, 0)
            pltpu.sync_copy(rows, acc_sp.at[idx_d.at[j]], add=True)

            @pl.when(c == 0)
            def _():
                pltpu.sync_copy(ev.at[pl.ds(eb, 128)],
                                den_sp.at[idx_d.at[j]], add=True)
        return 0

    lax.fori_loop(0, SUPER, _super, 0)
    plsc.subcore_barrier()

    off = pl.multiple_of(s * NSLICE, 128)

    @pl.when(c == 0)
    def _():
        pltpu.sync_copy(acc_sp.at[pl.ds(off, NSLICE), :],
                        out_lo.at[pl.ds(off, NSLICE), :])
        pltpu.sync_copy(den_sp.at[pl.ds(off, NSLICE)],
                        den_o.at[pl.ds(off, NSLICE)])

    @pl.when(c == 1)
    def _():
        pltpu.sync_copy(acc_sp.at[pl.ds(off, NSLICE), :],
                        out_hi.at[pl.ds(off, NSLICE), :])


_sc_l12 = pl.kernel(
    _sc_l12_body,
    out_type=(
        jax.ShapeDtypeStruct((NPAD, 32), F32),
        jax.ShapeDtypeStruct((NPAD, 32), F32),
        jax.ShapeDtypeStruct((NPAD,), F32),
    ),
    mesh=_MESH,
    compiler_params=_SC_PARAMS,
    scratch_types=[
        pltpu.VMEM_SHARED((NPAD, 32), F32),
        pltpu.VMEM_SHARED((NPAD,), F32),
        pltpu.VMEM((16, 128), jnp.int32),
        pltpu.VMEM((16, 128), jnp.int32),
        pltpu.VMEM((2048,), F32),
        pltpu.VMEM((2048,), F32),
        pltpu.VMEM((2048,), F32),
        pltpu.VMEM((128, 32), F32),
        pltpu.VMEM((16,), F32),
        pltpu.VMEM((128, 32), F32),
        pltpu.VMEM((128,), F32),
        pltpu.SemaphoreType.DMA,
        pltpu.SemaphoreType.DMA,
    ],
)


def _sc_l0_body(src_h, dst_h, asrc_h, adst_h, m_h, xa_h,
                out_a, out_b,
                acc_sp, idx_s, idx_d, av, dv, ev, rows, mv, zr,
                sem, sem2):
    c = lax.axis_index("c")
    s = lax.axis_index("s")

    zero16 = jnp.zeros((16,), F32)

    def _zr_body(r, _):
        zr[r, pl.ds(0, 16)] = zero16
        return 0

    lax.fori_loop(0, 128, _zr_body, 0)
    base = s * NSLICE
    for k in range(NSLICE // 128):
        off = pl.multiple_of(base + k * 128, 128)
        pltpu.sync_copy(zr, acc_sp.at[pl.ds(off, 128), :])
    plsc.subcore_barrier()

    pltpu.sync_copy(m_h, mv)
    mvec = mv[...]

    def _super(t, _):
        row0 = c * (ROWS // 2) + s * ROWS_PER_SUB0 + t * 16
        pltpu.sync_copy(src_h.at[pl.ds(row0, 16), :], idx_s)
        pltpu.sync_copy(dst_h.at[pl.ds(row0, 16), :], idx_d)
        descs = []
        for j in range(16):
            descs.append(
                pltpu.async_copy(asrc_h.at[idx_s.at[j]],
                                 av.at[pl.ds(j * 128, 128)], sem))
            descs.append(
                pltpu.async_copy(adst_h.at[idx_d.at[j]],
                                 dv.at[pl.ds(j * 128, 128)], sem))
        for dsc in descs:
            dsc.wait()

        def _e_body(g, _):
            z = av[pl.ds(g * 16, 16)] + dv[pl.ds(g * 16, 16)]
            l = jnp.maximum(z, F32(0.2) * z)
            ev[pl.ds(g * 16, 16)] = jnp.exp(l - mvec)
            return 0

        lax.fori_loop(0, 128, _e_body, 0)

        for j in range(16):
            pltpu.async_copy(xa_h.at[idx_s.at[j]], rows, sem2).wait()
            eb = j * 128

            def _sc_body(g, _):
                e16 = ev[pl.ds(eb + g * 16, 16)]
                r0 = g * 16
                for k in range(16):
                    scl = lax.broadcast(e16[k], (16,))
                    rows[r0 + k, pl.ds(0, 16)] = (
                        rows[r0 + k, pl.ds(0, 16)] * scl)
                return 0

            lax.fori_loop(0, 8, _sc_body, 0)
            pltpu.sync_copy(rows, acc_sp.at[idx_d.at[j]], add=True)
        return 0

    lax.fori_loop(0, SUPER0, _super, 0)
    plsc.subcore_barrier()

    off = pl.multiple_of(s * NSLICE, 128)

    @pl.when(c == 0)
    def _():
        pltpu.sync_copy(acc_sp.at[pl.ds(off, NSLICE), :],
                        out_a.at[pl.ds(off, NSLICE), :])

    @pl.when(c == 1)
    def _():
        pltpu.sync_copy(acc_sp.at[pl.ds(off, NSLICE), :],
                        out_b.at[pl.ds(off, NSLICE), :])


_sc_l0 = pl.kernel(
    _sc_l0_body,
    out_type=(
        jax.ShapeDtypeStruct((NPAD, 16), F32),
        jax.ShapeDtypeStruct((NPAD, 16), F32),
    ),
    mesh=_MESH,
    compiler_params=_SC_PARAMS,
    scratch_types=[
        pltpu.VMEM_SHARED((NPAD, 16), F32),
        pltpu.VMEM((16, 128), jnp.int32),
        pltpu.VMEM((16, 128), jnp.int32),
        pltpu.VMEM((2048,), F32),
        pltpu.VMEM((2048,), F32),
        pltpu.VMEM((2048,), F32),
        pltpu.VMEM((128, 16), F32),
        pltpu.VMEM((16,), F32),
        pltpu.VMEM((128, 16), F32),
        pltpu.SemaphoreType.DMA,
        pltpu.SemaphoreType.DMA,
    ],
)


def kernel(pos, edge_index, W0, as0, ad0, b0, W1, as1, ad1, b1,
           W2, as2, ad2, b2):
    loop = jnp.arange(N, dtype=jnp.int32)
    src = jnp.concatenate([edge_index[0].astype(jnp.int32), loop])
    dst = jnp.concatenate([edge_index[1].astype(jnp.int32), loop])
    pad = jnp.full((EPAD - E,), N, jnp.int32)
    src2 = jnp.concatenate([src, pad]).reshape(ROWS, 128)
    dst2 = jnp.concatenate([dst, pad]).reshape(ROWS, 128)
    pos4 = jnp.concatenate([pos, jnp.ones((N, 1), F32)], axis=1)
    pos4 = jnp.pad(pos4, ((0, NPAD - N), (0, 0)))

    xa0, asrc0, adst0, m0 = _tc_a(
        pos4, W0, as0.reshape(64, 1), ad0.reshape(64, 1))
    ag0a, ag0b = _sc_l0(src2, dst2, asrc0.reshape(-1), adst0.reshape(-1),
                        m0.reshape(-1)[:16], xa0)

    xlo1, xhi1, asrc1, adst1, m1 = _tc_b(
        ag0a, ag0b, W0, b0.reshape(1, 64), W1,
        as1.reshape(64, 1), ad1.reshape(64, 1))
    ag1lo, ag1hi, den1 = _sc_l12(
        src2, dst2, asrc1.reshape(-1), adst1.reshape(-1),
        m1.reshape(-1)[:16], xlo1, xhi1)

    xlo2, xhi2, asrc2, adst2, m2 = _tc_c(
        ag1lo, ag1hi, den1.reshape(NPAD, 1), W1, b1.reshape(1, 64), W2,
        as2.reshape(68, 1), ad2.reshape(68, 1))
    ag2lo, ag2hi, den2 = _sc_l12(
        src2, dst2, asrc2.reshape(-1), adst2.reshape(-1),
        m2.reshape(-1)[:16], xlo2, xhi2)

    (out,) = _tc_d(ag2lo, ag2hi, den2.reshape(NPAD, 1), W2,
                   b2.reshape(1, 68))
    return out[:N]


# speedup vs baseline: 51.3775x; 1.4086x over previous
import functools

import jax
import jax.numpy as jnp
from jax import lax
from jax.experimental import pallas as pl
from jax.experimental.pallas import tpu as pltpu
from jax.experimental.pallas import tpu_sc as plsc

F32 = jnp.float32

N = 50000
NPAD = 51200
E = 850000
EPAD = 851968
ROWS = EPAD // 128
NC, NS = 2, 16
ROWS_PER_SUB = ROWS // NS
SUPER = ROWS_PER_SUB // 16
ROWS_PER_SUB0 = ROWS // (NC * NS)
SUPER0 = ROWS_PER_SUB0 // 16
NSLICE = NPAD // NS
SPN = 50048
SPSLICE = SPN // NS
BLK = 2048
GRID = NPAD // BLK
NEG = -3.0e38
EPS = 1e-16

_MESH = plsc.VectorSubcoreMesh(core_axis_name="c", subcore_axis_name="s",
                               num_cores=NC, num_subcores=NS)
_SC_PARAMS = pltpu.CompilerParams(use_tc_tiling_on_sc=False)


def _tc_a_body(pos4_ref, w0_ref, as0_ref, ad0_ref,
               xa0_ref, asrc_ref, adst_ref, m_ref, msum_ref):
    i = pl.program_id(0)

    @pl.when(i == 0)
    def _():
        msum_ref[0] = NEG
        msum_ref[1] = NEG

    p4 = pos4_ref[...]
    x3 = p4[:, 0:3]
    was = w0_ref[...] @ as0_ref[...]
    wad = w0_ref[...] @ ad0_ref[...]
    asrc = x3 @ was
    adst = x3 @ wad
    xa0_ref[...] = jnp.concatenate(
        [p4, jnp.zeros((BLK, 12), F32)], axis=1)
    asrc_ref[...] = asrc
    adst_ref[...] = adst
    msum_ref[0] = jnp.maximum(msum_ref[0], jnp.max(asrc))
    msum_ref[1] = jnp.maximum(msum_ref[1], jnp.max(adst))

    @pl.when(i == GRID - 1)
    def _():
        z = msum_ref[0] + msum_ref[1]
        m = jnp.maximum(z, F32(0.2) * z)
        m_ref[...] = jnp.full((8, 128), m, F32)


_tc_a = pl.pallas_call(
    _tc_a_body,
    grid=(GRID,),
    in_specs=[
        pl.BlockSpec((BLK, 4), lambda i: (i, 0)),
        pl.BlockSpec((3, 64), lambda i: (0, 0)),
        pl.BlockSpec((64, 1), lambda i: (0, 0)),
        pl.BlockSpec((64, 1), lambda i: (0, 0)),
    ],
    out_specs=[
        pl.BlockSpec((BLK, 16), lambda i: (i, 0)),
        pl.BlockSpec((BLK, 1), lambda i: (i, 0)),
        pl.BlockSpec((BLK, 1), lambda i: (i, 0)),
        pl.BlockSpec((8, 128), lambda i: (0, 0)),
    ],
    out_shape=[
        jax.ShapeDtypeStruct((NPAD, 16), F32),
        jax.ShapeDtypeStruct((NPAD, 1), F32),
        jax.ShapeDtypeStruct((NPAD, 1), F32),
        jax.ShapeDtypeStruct((8, 128), F32),
    ],
    scratch_shapes=[pltpu.SMEM((2,), F32)],
)


def _tc_b_body(aga_ref, agb_ref, w0_ref, b0_ref, w1_ref, as1_ref, ad1_ref,
               xlo_ref, xhi_ref, asrc_ref, adst_ref, m_ref, msum_ref):
    i = pl.program_id(0)

    @pl.when(i == 0)
    def _():
        msum_ref[0] = NEG
        msum_ref[1] = NEG

    agg = aga_ref[...] + agb_ref[...]
    den = agg[:, 3:4] + EPS
    xn = agg[:, 0:3] / den
    x1 = jnp.maximum(xn @ w0_ref[...] + b0_ref[...], 0.0)
    xlo_ref[...] = x1[:, 0:32]
    xhi_ref[...] = x1[:, 32:64]
    was = w1_ref[...] @ as1_ref[...]
    wad = w1_ref[...] @ ad1_ref[...]
    asrc = x1 @ was
    adst = x1 @ wad
    asrc_ref[...] = asrc
    adst_ref[...] = adst
    msum_ref[0] = jnp.maximum(msum_ref[0], jnp.max(asrc))
    msum_ref[1] = jnp.maximum(msum_ref[1], jnp.max(adst))

    @pl.when(i == GRID - 1)
    def _():
        z = msum_ref[0] + msum_ref[1]
        m = jnp.maximum(z, F32(0.2) * z)
        m_ref[...] = jnp.full((8, 128), m, F32)


_tc_b = pl.pallas_call(
    _tc_b_body,
    grid=(GRID,),
    in_specs=[
        pl.BlockSpec((BLK, 16), lambda i: (i, 0)),
        pl.BlockSpec((BLK, 16), lambda i: (i, 0)),
        pl.BlockSpec((3, 64), lambda i: (0, 0)),
        pl.BlockSpec((1, 64), lambda i: (0, 0)),
        pl.BlockSpec((64, 64), lambda i: (0, 0)),
        pl.BlockSpec((64, 1), lambda i: (0, 0)),
        pl.BlockSpec((64, 1), lambda i: (0, 0)),
    ],
    out_specs=[
        pl.BlockSpec((BLK, 32), lambda i: (i, 0)),
        pl.BlockSpec((BLK, 32), lambda i: (i, 0)),
        pl.BlockSpec((BLK, 1), lambda i: (i, 0)),
        pl.BlockSpec((BLK, 1), lambda i: (i, 0)),
        pl.BlockSpec((8, 128), lambda i: (0, 0)),
    ],
    out_shape=[
        jax.ShapeDtypeStruct((NPAD, 32), F32),
        jax.ShapeDtypeStruct((NPAD, 32), F32),
        jax.ShapeDtypeStruct((NPAD, 1), F32),
        jax.ShapeDtypeStruct((NPAD, 1), F32),
        jax.ShapeDtypeStruct((8, 128), F32),
    ],
    scratch_shapes=[pltpu.SMEM((2,), F32)],
)


def _tc_c_body(aglo_ref, aghi_ref, den_ref, w1_ref, b1_ref, w2_ref,
               as2_ref, ad2_ref,
               xlo_ref, xhi_ref, asrc_ref, adst_ref, m_ref, msum_ref):
    i = pl.program_id(0)

    @pl.when(i == 0)
    def _():
        msum_ref[0] = NEG
        msum_ref[1] = NEG

    inv = 1.0 / (den_ref[...] + EPS)
    xn_lo = aglo_ref[...] * inv
    xn_hi = aghi_ref[...] * inv
    w1 = w1_ref[...]
    x2 = jnp.maximum(
        xn_lo @ w1[0:32, :] + xn_hi @ w1[32:64, :] + b1_ref[...], 0.0)
    xlo_ref[...] = x2[:, 0:32]
    xhi_ref[...] = x2[:, 32:64]
    was = w2_ref[...] @ as2_ref[...]
    wad = w2_ref[...] @ ad2_ref[...]
    asrc = x2 @ was
    adst = x2 @ wad
    asrc_ref[...] = asrc
    adst_ref[...] = adst
    msum_ref[0] = jnp.maximum(msum_ref[0], jnp.max(asrc))
    msum_ref[1] = jnp.maximum(msum_ref[1], jnp.max(adst))

    @pl.when(i == GRID - 1)
    def _():
        z = msum_ref[0] + msum_ref[1]
        m = jnp.maximum(z, F32(0.2) * z)
        m_ref[...] = jnp.full((8, 128), m, F32)


_tc_c = pl.pallas_call(
    _tc_c_body,
    grid=(GRID,),
    in_specs=[
        pl.BlockSpec((BLK, 32), lambda i: (i, 0)),
        pl.BlockSpec((BLK, 32), lambda i: (i, 0)),
        pl.BlockSpec((BLK, 1), lambda i: (i, 0)),
        pl.BlockSpec((64, 64), lambda i: (0, 0)),
        pl.BlockSpec((1, 64), lambda i: (0, 0)),
        pl.BlockSpec((64, 68), lambda i: (0, 0)),
        pl.BlockSpec((68, 1), lambda i: (0, 0)),
        pl.BlockSpec((68, 1), lambda i: (0, 0)),
    ],
    out_specs=[
        pl.BlockSpec((BLK, 32), lambda i: (i, 0)),
        pl.BlockSpec((BLK, 32), lambda i: (i, 0)),
        pl.BlockSpec((BLK, 1), lambda i: (i, 0)),
        pl.BlockSpec((BLK, 1), lambda i: (i, 0)),
        pl.BlockSpec((8, 128), lambda i: (0, 0)),
    ],
    out_shape=[
        jax.ShapeDtypeStruct((NPAD, 32), F32),
        jax.ShapeDtypeStruct((NPAD, 32), F32),
        jax.ShapeDtypeStruct((NPAD, 1), F32),
        jax.ShapeDtypeStruct((NPAD, 1), F32),
        jax.ShapeDtypeStruct((8, 128), F32),
    ],
    scratch_shapes=[pltpu.SMEM((2,), F32)],
)


def _tc_d_body(aglo_ref, aghi_ref, den_ref, w2_ref, b2_ref, out_ref):
    inv = 1.0 / (den_ref[...] + EPS)
    xn_lo = aglo_ref[...] * inv
    xn_hi = aghi_ref[...] * inv
    w2 = w2_ref[...]
    out_ref[...] = xn_lo @ w2[0:32, :] + xn_hi @ w2[32:64, :] + b2_ref[...]


_tc_d = pl.pallas_call(
    _tc_d_body,
    grid=(GRID,),
    in_specs=[
        pl.BlockSpec((BLK, 32), lambda i: (i, 0)),
        pl.BlockSpec((BLK, 32), lambda i: (i, 0)),
        pl.BlockSpec((BLK, 1), lambda i: (i, 0)),
        pl.BlockSpec((64, 68), lambda i: (0, 0)),
        pl.BlockSpec((1, 68), lambda i: (0, 0)),
    ],
    out_specs=[pl.BlockSpec((BLK, 68), lambda i: (i, 0))],
    out_shape=[jax.ShapeDtypeStruct((NPAD, 68), F32)],
)


def _make_sc_body(nv, feature_split, nsuper):
    def body(src_h, dst_h, asrc_h, adst_h, m_h, xlo_h, xhi_h,
             out_lo, out_hi, den_o,
             acc_sp, den_sp, idx_s, idx_d, av, ev,
             rows_a, rows_b, rows_c, mv, zd,
             sem_ga, sem_gr0, sem_gr1, sem_gr2, sem_sc, sem_den):
        c = lax.axis_index("c")
        s = lax.axis_index("s")
        rows3 = (rows_a, rows_b, rows_c)
        gr_sems = (sem_gr0, sem_gr1, sem_gr2)
        zero16 = jnp.zeros((16,), F32)

        def _zr_body(r, _):
            for v in range(nv):
                rows_a[r, pl.ds(16 * v, 16)] = zero16
            return 0

        lax.fori_loop(0, 128, _zr_body, 0)
        for k in range(8):
            zd[pl.ds(16 * k, 16)] = zero16
        base = s * SPSLICE
        for k in range(24):
            off = pl.multiple_of(base + k * 128, 8)
            pltpu.sync_copy(rows_a, acc_sp.at[pl.ds(off, 128), :])
            if feature_split:
                pltpu.sync_copy(zd, den_sp.at[pl.ds(off, 128)])
        offt = pl.multiple_of(base + 24 * 128, 8)
        pltpu.sync_copy(rows_a.at[pl.ds(0, 56), :],
                        acc_sp.at[pl.ds(offt, 56), :])
        if feature_split:
            pltpu.sync_copy(zd.at[pl.ds(0, 56)], den_sp.at[pl.ds(offt, 56)])

        @pl.when(s < (NPAD - SPN) // 128)
        def _():
            offh = pl.multiple_of(SPN + s * 128, 8)

            @pl.when(c == 0)
            def _():
                pltpu.sync_copy(rows_a, out_lo.at[pl.ds(offh, 128), :])
                pltpu.sync_copy(zd, den_o.at[pl.ds(offh, 128)])

            @pl.when(c == 1)
            def _():
                pltpu.sync_copy(rows_a, out_hi.at[pl.ds(offh, 128), :])
        plsc.subcore_barrier()

        pltpu.sync_copy(m_h, mv)
        mvec = mv[...]

        if feature_split:
            sub_rows = ROWS_PER_SUB
            row_base = s * sub_rows
        else:
            sub_rows = ROWS_PER_SUB0
            row_base = c * (ROWS // 2) + s * sub_rows

        def _load_idx(t, pb):
            row0 = jnp.minimum(row_base + t * 16, ROWS - 16)
            pltpu.sync_copy(src_h.at[pl.ds(row0, 16), :],
                            idx_s.at[pl.ds(pb, 16), :])
            pltpu.sync_copy(dst_h.at[pl.ds(row0, 16), :],
                            idx_d.at[pl.ds(pb, 16), :])

        def _fire_a(pb, eb):
            descs = []
            for j in range(16):
                o = pl.multiple_of(eb + j * 128, 128)
                descs.append(pltpu.async_copy(
                    asrc_h.at[idx_s.at[pb + j]],
                    av.at[pl.ds(o - eb, 128)], sem_ga))
                descs.append(pltpu.async_copy(
                    adst_h.at[idx_d.at[pb + j]],
                    ev.at[pl.ds(o, 128)], sem_ga))
            return descs

        def _compute_e(eb):
            def _e_body(g, _):
                o = pl.multiple_of(eb + g * 16, 16)
                z = av[pl.ds(pl.multiple_of(g * 16, 16), 16)] + ev[pl.ds(o, 16)]
                l = jnp.maximum(z, 0.2 * z)
                ev[pl.ds(o, 16)] = jnp.exp(l - mvec)
                return 0

            lax.fori_loop(0, 128, _e_body, 0)

        def _fire_rows(pb, j, b):
            dst = rows3[b]
            if feature_split:
                @pl.when(c == 0)
                def _():
                    pltpu.async_copy(xlo_h.at[idx_s.at[pb + j]],
                                     dst, gr_sems[b])

                @pl.when(c == 1)
                def _():
                    pltpu.async_copy(xhi_h.at[idx_s.at[pb + j]],
                                     dst, gr_sems[b])
                return pltpu.make_async_copy(
                    xlo_h.at[idx_s.at[pb + j]], dst, gr_sems[b])
            return pltpu.async_copy(xlo_h.at[idx_s.at[pb + j]],
                                    dst, gr_sems[b])

        _load_idx(0, 0)
        for d in _fire_a(0, 0):
            d.wait()
        _compute_e(0)

        def _one_super(t, p):
            pb = p * 16
            pbn = (1 - p) * 16
            eb = p * 2048
            ebn = (1 - p) * 2048

            gdesc = {0: _fire_rows(pb, 0, 0), 1: _fire_rows(pb, 1, 1)}
            _load_idx(t + 1, pbn)
            a_descs = _fire_a(pbn, ebn)

            sdesc = {}
            den_descs = []
            for j in range(16):
                b = j % 3
                gdesc[j].wait()
                if j >= 1:
                    sdesc[j - 1].wait()
                eb_j = eb + j * 128

                def _scale(g, _):
                    e16 = ev[pl.ds(pl.multiple_of(eb_j + g * 16, 16), 16)]
                    r0 = g * 16
                    for k in range(16):
                        scl = lax.broadcast(e16[k], (16,))
                        for v in range(nv):
                            rows3[b][r0 + k, pl.ds(16 * v, 16)] = (
                                rows3[b][r0 + k, pl.ds(16 * v, 16)] * scl)
                    return 0

                lax.fori_loop(0, 8, _scale, 0)
                sdesc[j] = pltpu.async_copy(
                    rows3[b], acc_sp.at[idx_d.at[pb + j]], sem_sc, add=True)
                if feature_split:
                    @pl.when(c == 0)
                    def _():
                        pltpu.async_copy(ev.at[pl.ds(eb_j, 128)],
                                         den_sp.at[idx_d.at[pb + j]],
                                         sem_den, add=True)
                    den_descs.append(pltpu.make_async_copy(
                        ev.at[pl.ds(eb_j, 128)],
                        den_sp.at[idx_d.at[pb + j]], sem_den))
                if j + 2 < 16:
                    gdesc[j + 2] = _fire_rows(pb, j + 2, (j + 2) % 3)

            for d in a_descs:
                d.wait()
            _compute_e(ebn)
            sdesc[15].wait()
            if feature_split:
                @pl.when(c == 0)
                def _():
                    for d in den_descs:
                        d.wait()

        def _super_pair(tp, _):
            _one_super(tp * 2, 0)
            _one_super(tp * 2 + 1, 1)
            return 0

        lax.fori_loop(0, nsuper // 2, _super_pair, 0)
        if nsuper % 2:
            _one_super(nsuper - 1, 0)
        plsc.subcore_barrier()

        off = pl.multiple_of(s * SPSLICE, 8)

        @pl.when(c == 0)
        def _():
            pltpu.sync_copy(acc_sp.at[pl.ds(off, SPSLICE), :],
                            out_lo.at[pl.ds(off, SPSLICE), :])
            if feature_split:
                pltpu.sync_copy(den_sp.at[pl.ds(off, SPSLICE)],
                                den_o.at[pl.ds(off, SPSLICE)])

        @pl.when(c == 1)
        def _():
            pltpu.sync_copy(acc_sp.at[pl.ds(off, SPSLICE), :],
                            out_hi.at[pl.ds(off, SPSLICE), :])

    return body


def _make_sc_kernel(nv, feature_split, nsuper):
    width = 16 * nv
    out_type = (
        jax.ShapeDtypeStruct((NPAD, width), F32),
        jax.ShapeDtypeStruct((NPAD, width), F32),
        jax.ShapeDtypeStruct((NPAD,), F32),
    )
    scratch = [
        pltpu.VMEM_SHARED((SPN, width), F32),
        pltpu.VMEM_SHARED((SPN,), F32),
        pltpu.VMEM((32, 128), jnp.int32),
        pltpu.VMEM((32, 128), jnp.int32),
        pltpu.VMEM((2048,), F32),
        pltpu.VMEM((4096,), F32),
        pltpu.VMEM((128, width), F32),
        pltpu.VMEM((128, width), F32),
        pltpu.VMEM((128, width), F32),
        pltpu.VMEM((16,), F32),
        pltpu.VMEM((128,), F32),
        pltpu.SemaphoreType.DMA,
        pltpu.SemaphoreType.DMA,
        pltpu.SemaphoreType.DMA,
        pltpu.SemaphoreType.DMA,
        pltpu.SemaphoreType.DMA,
        pltpu.SemaphoreType.DMA,
    ]
    return pl.kernel(
        _make_sc_body(nv, feature_split, nsuper),
        out_type=out_type,
        mesh=_MESH,
        compiler_params=_SC_PARAMS,
        scratch_types=scratch,
    )


_sc_l12 = _make_sc_kernel(2, True, SUPER)
_sc_l0 = _make_sc_kernel(1, False, SUPER0)


def kernel(pos, edge_index, W0, as0, ad0, b0, W1, as1, ad1, b1,
           W2, as2, ad2, b2):
    loop = jnp.arange(N, dtype=jnp.int32)
    src = jnp.concatenate([edge_index[0].astype(jnp.int32), loop])
    dst = jnp.concatenate([edge_index[1].astype(jnp.int32), loop])
    pad = jnp.full((EPAD - E,), N, jnp.int32)
    src2 = jnp.concatenate([src, pad]).reshape(ROWS, 128)
    dst2 = jnp.concatenate([dst, pad]).reshape(ROWS, 128)
    pos4 = jnp.concatenate([pos, jnp.ones((N, 1), F32)], axis=1)
    pos4 = jnp.pad(pos4, ((0, NPAD - N), (0, 0)))

    xa0, asrc0, adst0, m0 = _tc_a(
        pos4, W0, as0.reshape(64, 1), ad0.reshape(64, 1))
    ag0a, ag0b, _unused_den0 = _sc_l0(
        src2, dst2, asrc0.reshape(-1), adst0.reshape(-1),
        m0.reshape(-1)[:16], xa0, xa0)

    xlo1, xhi1, asrc1, adst1, m1 = _tc_b(
        ag0a, ag0b, W0, b0.reshape(1, 64), W1,
        as1.reshape(64, 1), ad1.reshape(64, 1))
    ag1lo, ag1hi, den1 = _sc_l12(
        src2, dst2, asrc1.reshape(-1), adst1.reshape(-1),
        m1.reshape(-1)[:16], xlo1, xhi1)

    xlo2, xhi2, asrc2, adst2, m2 = _tc_c(
        ag1lo, ag1hi, den1.reshape(NPAD, 1), W1, b1.reshape(1, 64), W2,
        as2.reshape(68, 1), ad2.reshape(68, 1))
    ag2lo, ag2hi, den2 = _sc_l12(
        src2, dst2, asrc2.reshape(-1), adst2.reshape(-1),
        m2.reshape(-1)[:16], xlo2, xhi2)

    (out,) = _tc_d(ag2lo, ag2hi, den2.reshape(NPAD, 1), W2,
                   b2.reshape(1, 68))
    return out[:N]


# speedup vs baseline: 54.8111x; 1.0668x over previous
import functools

import jax
import jax.numpy as jnp
from jax import lax
from jax.experimental import pallas as pl
from jax.experimental.pallas import tpu as pltpu
from jax.experimental.pallas import tpu_sc as plsc

F32 = jnp.float32

N = 50000
NPAD = 51200
E = 850000
EPAD = 851968
ROWS = EPAD // 128
NC, NS = 2, 16
ROWS_PER_SUB = ROWS // NS
SUPER = ROWS_PER_SUB // 16
ROWS_PER_SUB0 = ROWS // (NC * NS)
SUPER0 = ROWS_PER_SUB0 // 16
NSLICE = NPAD // NS
SPN = 50048
SPSLICE = SPN // NS
BLK = 2048
GRID = NPAD // BLK
NEG = -3.0e38
EPS = 1e-16

_MESH = plsc.VectorSubcoreMesh(core_axis_name="c", subcore_axis_name="s",
                               num_cores=NC, num_subcores=NS)
_SC_PARAMS = pltpu.CompilerParams(use_tc_tiling_on_sc=False)


def _tc_a_body(pos4_ref, w0_ref, as0_ref, ad0_ref,
               xa0_ref, asrc_ref, adst_ref, m_ref, msum_ref):
    i = pl.program_id(0)

    @pl.when(i == 0)
    def _():
        msum_ref[0] = NEG
        msum_ref[1] = NEG

    p4 = pos4_ref[...]
    x3 = p4[:, 0:3]
    was = w0_ref[...] @ as0_ref[...]
    wad = w0_ref[...] @ ad0_ref[...]
    asrc = x3 @ was
    adst = x3 @ wad
    xa0_ref[...] = jnp.concatenate(
        [p4, jnp.zeros((BLK, 12), F32)], axis=1)
    asrc_ref[...] = asrc
    adst_ref[...] = adst
    msum_ref[0] = jnp.maximum(msum_ref[0], jnp.max(asrc))
    msum_ref[1] = jnp.maximum(msum_ref[1], jnp.max(adst))

    @pl.when(i == GRID - 1)
    def _():
        z = msum_ref[0] + msum_ref[1]
        m = jnp.maximum(z, F32(0.2) * z)
        m_ref[...] = jnp.full((8, 128), m, F32)


_tc_a = pl.pallas_call(
    _tc_a_body,
    grid=(GRID,),
    in_specs=[
        pl.BlockSpec((BLK, 4), lambda i: (i, 0)),
        pl.BlockSpec((3, 64), lambda i: (0, 0)),
        pl.BlockSpec((64, 1), lambda i: (0, 0)),
        pl.BlockSpec((64, 1), lambda i: (0, 0)),
    ],
    out_specs=[
        pl.BlockSpec((BLK, 16), lambda i: (i, 0)),
        pl.BlockSpec((BLK, 1), lambda i: (i, 0)),
        pl.BlockSpec((BLK, 1), lambda i: (i, 0)),
        pl.BlockSpec((8, 128), lambda i: (0, 0)),
    ],
    out_shape=[
        jax.ShapeDtypeStruct((NPAD, 16), F32),
        jax.ShapeDtypeStruct((NPAD, 1), F32),
        jax.ShapeDtypeStruct((NPAD, 1), F32),
        jax.ShapeDtypeStruct((8, 128), F32),
    ],
    scratch_shapes=[pltpu.SMEM((2,), F32)],
)


def _tc_b_body(aga_ref, agb_ref, w0_ref, b0_ref, w1_ref, as1_ref, ad1_ref,
               xlo_ref, xhi_ref, asrc_ref, adst_ref, m_ref, msum_ref):
    i = pl.program_id(0)

    @pl.when(i == 0)
    def _():
        msum_ref[0] = NEG
        msum_ref[1] = NEG

    agg = aga_ref[...] + agb_ref[...]
    den = agg[:, 3:4] + EPS
    xn = agg[:, 0:3] / den
    x1 = jnp.maximum(xn @ w0_ref[...] + b0_ref[...], 0.0)
    xlo_ref[...] = x1[:, 0:32]
    xhi_ref[...] = x1[:, 32:64]
    was = w1_ref[...] @ as1_ref[...]
    wad = w1_ref[...] @ ad1_ref[...]
    asrc = x1 @ was
    adst = x1 @ wad
    asrc_ref[...] = asrc
    adst_ref[...] = adst
    msum_ref[0] = jnp.maximum(msum_ref[0], jnp.max(asrc))
    msum_ref[1] = jnp.maximum(msum_ref[1], jnp.max(adst))

    @pl.when(i == GRID - 1)
    def _():
        z = msum_ref[0] + msum_ref[1]
        m = jnp.maximum(z, F32(0.2) * z)
        m_ref[...] = jnp.full((8, 128), m, F32)


_tc_b = pl.pallas_call(
    _tc_b_body,
    grid=(GRID,),
    in_specs=[
        pl.BlockSpec((BLK, 16), lambda i: (i, 0)),
        pl.BlockSpec((BLK, 16), lambda i: (i, 0)),
        pl.BlockSpec((3, 64), lambda i: (0, 0)),
        pl.BlockSpec((1, 64), lambda i: (0, 0)),
        pl.BlockSpec((64, 64), lambda i: (0, 0)),
        pl.BlockSpec((64, 1), lambda i: (0, 0)),
        pl.BlockSpec((64, 1), lambda i: (0, 0)),
    ],
    out_specs=[
        pl.BlockSpec((BLK, 32), lambda i: (i, 0)),
        pl.BlockSpec((BLK, 32), lambda i: (i, 0)),
        pl.BlockSpec((BLK, 1), lambda i: (i, 0)),
        pl.BlockSpec((BLK, 1), lambda i: (i, 0)),
        pl.BlockSpec((8, 128), lambda i: (0, 0)),
    ],
    out_shape=[
        jax.ShapeDtypeStruct((NPAD, 32), F32),
        jax.ShapeDtypeStruct((NPAD, 32), F32),
        jax.ShapeDtypeStruct((NPAD, 1), F32),
        jax.ShapeDtypeStruct((NPAD, 1), F32),
        jax.ShapeDtypeStruct((8, 128), F32),
    ],
    scratch_shapes=[pltpu.SMEM((2,), F32)],
)


def _tc_c_body(aglo_ref, aghi_ref, den_ref, w1_ref, b1_ref, w2_ref,
               as2_ref, ad2_ref,
               xlo_ref, xhi_ref, asrc_ref, adst_ref, m_ref, msum_ref):
    i = pl.program_id(0)

    @pl.when(i == 0)
    def _():
        msum_ref[0] = NEG
        msum_ref[1] = NEG

    inv = 1.0 / (den_ref[...] + EPS)
    xn_lo = aglo_ref[...] * inv
    xn_hi = aghi_ref[...] * inv
    w1 = w1_ref[...]
    x2 = jnp.maximum(
        xn_lo @ w1[0:32, :] + xn_hi @ w1[32:64, :] + b1_ref[...], 0.0)
    xlo_ref[...] = x2[:, 0:32]
    xhi_ref[...] = x2[:, 32:64]
    was = w2_ref[...] @ as2_ref[...]
    wad = w2_ref[...] @ ad2_ref[...]
    asrc = x2 @ was
    adst = x2 @ wad
    asrc_ref[...] = asrc
    adst_ref[...] = adst
    msum_ref[0] = jnp.maximum(msum_ref[0], jnp.max(asrc))
    msum_ref[1] = jnp.maximum(msum_ref[1], jnp.max(adst))

    @pl.when(i == GRID - 1)
    def _():
        z = msum_ref[0] + msum_ref[1]
        m = jnp.maximum(z, F32(0.2) * z)
        m_ref[...] = jnp.full((8, 128), m, F32)


_tc_c = pl.pallas_call(
    _tc_c_body,
    grid=(GRID,),
    in_specs=[
        pl.BlockSpec((BLK, 32), lambda i: (i, 0)),
        pl.BlockSpec((BLK, 32), lambda i: (i, 0)),
        pl.BlockSpec((BLK, 1), lambda i: (i, 0)),
        pl.BlockSpec((64, 64), lambda i: (0, 0)),
        pl.BlockSpec((1, 64), lambda i: (0, 0)),
        pl.BlockSpec((64, 68), lambda i: (0, 0)),
        pl.BlockSpec((68, 1), lambda i: (0, 0)),
        pl.BlockSpec((68, 1), lambda i: (0, 0)),
    ],
    out_specs=[
        pl.BlockSpec((BLK, 32), lambda i: (i, 0)),
        pl.BlockSpec((BLK, 32), lambda i: (i, 0)),
        pl.BlockSpec((BLK, 1), lambda i: (i, 0)),
        pl.BlockSpec((BLK, 1), lambda i: (i, 0)),
        pl.BlockSpec((8, 128), lambda i: (0, 0)),
    ],
    out_shape=[
        jax.ShapeDtypeStruct((NPAD, 32), F32),
        jax.ShapeDtypeStruct((NPAD, 32), F32),
        jax.ShapeDtypeStruct((NPAD, 1), F32),
        jax.ShapeDtypeStruct((NPAD, 1), F32),
        jax.ShapeDtypeStruct((8, 128), F32),
    ],
    scratch_shapes=[pltpu.SMEM((2,), F32)],
)


def _tc_d_body(aglo_ref, aghi_ref, den_ref, w2_ref, b2_ref, out_ref):
    inv = 1.0 / (den_ref[...] + EPS)
    xn_lo = aglo_ref[...] * inv
    xn_hi = aghi_ref[...] * inv
    w2 = w2_ref[...]
    out_ref[...] = xn_lo @ w2[0:32, :] + xn_hi @ w2[32:64, :] + b2_ref[...]


_tc_d = pl.pallas_call(
    _tc_d_body,
    grid=(GRID,),
    in_specs=[
        pl.BlockSpec((BLK, 32), lambda i: (i, 0)),
        pl.BlockSpec((BLK, 32), lambda i: (i, 0)),
        pl.BlockSpec((BLK, 1), lambda i: (i, 0)),
        pl.BlockSpec((64, 68), lambda i: (0, 0)),
        pl.BlockSpec((1, 68), lambda i: (0, 0)),
    ],
    out_specs=[pl.BlockSpec((BLK, 68), lambda i: (i, 0))],
    out_shape=[jax.ShapeDtypeStruct((NPAD, 68), F32)],
)


def _make_sc_body(nv, feature_split, nsuper):
    def body(src_h, dst_h, asrc_h, adst_h, m_h, xlo_h, xhi_h,
             out_lo, out_hi, den_o,
             acc_sp, den_sp, idx_s, idx_d, av, ev,
             rows_a, rows_b, rows_c, mv, zd,
             sem_ga, sem_gr0, sem_gr1, sem_gr2, sem_sc, sem_den, sem_idx):
        c = lax.axis_index("c")
        s = lax.axis_index("s")
        rows3 = (rows_a, rows_b, rows_c)
        gr_sems = (sem_gr0, sem_gr1, sem_gr2)
        zero16 = jnp.zeros((16,), F32)

        def _zr_body(r, _):
            for v in range(nv):
                rows_a[r, pl.ds(16 * v, 16)] = zero16
            return 0

        lax.fori_loop(0, 128, _zr_body, 0)
        for k in range(8):
            zd[pl.ds(16 * k, 16)] = zero16
        base = s * SPSLICE
        for k in range(24):
            off = pl.multiple_of(base + k * 128, 8)
            pltpu.sync_copy(rows_a, acc_sp.at[pl.ds(off, 128), :])
            if feature_split:
                pltpu.sync_copy(zd, den_sp.at[pl.ds(off, 128)])
        offt = pl.multiple_of(base + 24 * 128, 8)
        pltpu.sync_copy(rows_a.at[pl.ds(0, 56), :],
                        acc_sp.at[pl.ds(offt, 56), :])
        if feature_split:
            pltpu.sync_copy(zd.at[pl.ds(0, 56)], den_sp.at[pl.ds(offt, 56)])

        @pl.when(s < (NPAD - SPN) // 128)
        def _():
            offh = pl.multiple_of(SPN + s * 128, 8)

            @pl.when(c == 0)
            def _():
                pltpu.sync_copy(rows_a, out_lo.at[pl.ds(offh, 128), :])
                pltpu.sync_copy(zd, den_o.at[pl.ds(offh, 128)])

            @pl.when(c == 1)
            def _():
                pltpu.sync_copy(rows_a, out_hi.at[pl.ds(offh, 128), :])
        plsc.subcore_barrier()

        pltpu.sync_copy(m_h, mv)
        mvec = mv[...]

        if feature_split:
            sub_rows = ROWS_PER_SUB
            row_base = s * sub_rows
        else:
            sub_rows = ROWS_PER_SUB0
            row_base = c * (ROWS // 2) + s * sub_rows

        def _load_idx(t, pb):
            row0 = jnp.minimum(row_base + t * 16, ROWS - 16)
            pltpu.sync_copy(src_h.at[pl.ds(row0, 16), :],
                            idx_s.at[pl.ds(pb, 16), :])
            pltpu.sync_copy(dst_h.at[pl.ds(row0, 16), :],
                            idx_d.at[pl.ds(pb, 16), :])

        def _load_idx_async(t, pb):
            row0 = jnp.minimum(row_base + t * 16, ROWS - 16)
            return [
                pltpu.async_copy(src_h.at[pl.ds(row0, 16), :],
                                 idx_s.at[pl.ds(pb, 16), :], sem_idx),
                pltpu.async_copy(dst_h.at[pl.ds(row0, 16), :],
                                 idx_d.at[pl.ds(pb, 16), :], sem_idx),
            ]

        def _fire_a(pb, eb):
            descs = []
            for j in range(16):
                o = pl.multiple_of(eb + j * 128, 128)
                descs.append(pltpu.async_copy(
                    asrc_h.at[idx_s.at[pb + j]],
                    av.at[pl.ds(o - eb, 128)], sem_ga))
                descs.append(pltpu.async_copy(
                    adst_h.at[idx_d.at[pb + j]],
                    ev.at[pl.ds(o, 128)], sem_ga))
            return descs

        def _compute_e(eb):
            def _e_body(g, _):
                for u in range(4):
                    o = pl.multiple_of(eb + g * 64 + u * 16, 16)
                    z = (av[pl.ds(pl.multiple_of(g * 64 + u * 16, 16), 16)]
                         + ev[pl.ds(o, 16)])
                    l = jnp.maximum(z, 0.2 * z)
                    ev[pl.ds(o, 16)] = jnp.exp(l - mvec)
                return 0

            lax.fori_loop(0, 32, _e_body, 0)

        def _fire_rows(pb, j, b):
            dst = rows3[b]
            if feature_split:
                @pl.when(c == 0)
                def _():
                    pltpu.async_copy(xlo_h.at[idx_s.at[pb + j]],
                                     dst, gr_sems[b])

                @pl.when(c == 1)
                def _():
                    pltpu.async_copy(xhi_h.at[idx_s.at[pb + j]],
                                     dst, gr_sems[b])
                return pltpu.make_async_copy(
                    xlo_h.at[idx_s.at[pb + j]], dst, gr_sems[b])
            return pltpu.async_copy(xlo_h.at[idx_s.at[pb + j]],
                                    dst, gr_sems[b])

        _load_idx(0, 0)
        for d in _fire_a(0, 0):
            d.wait()
        _compute_e(0)

        def _one_super(t, p):
            pb = p * 16
            pbn = (1 - p) * 16
            eb = p * 2048
            ebn = (1 - p) * 2048

            gdesc = {0: _fire_rows(pb, 0, 0), 1: _fire_rows(pb, 1, 1)}
            idx_descs = _load_idx_async(t + 1, pbn)
            a_descs = []

            sdesc = {}
            den_descs = []
            for j in range(16):
                b = j % 3
                gdesc[j].wait()
                eb_j = eb + j * 128

                def _scale(g, _):
                    e16 = ev[pl.ds(pl.multiple_of(eb_j + g * 16, 16), 16)]
                    r0 = g * 16
                    for k in range(16):
                        scl = lax.broadcast(e16[k], (16,))
                        for v in range(nv):
                            rows3[b][r0 + k, pl.ds(16 * v, 16)] = (
                                rows3[b][r0 + k, pl.ds(16 * v, 16)] * scl)
                    return 0

                lax.fori_loop(0, 8, _scale, 0)
                if j == 0:
                    for d in idx_descs:
                        d.wait()
                    a_descs = _fire_a(pbn, ebn)
                if j >= 1:
                    sdesc[j - 1].wait()
                sdesc[j] = pltpu.async_copy(
                    rows3[b], acc_sp.at[idx_d.at[pb + j]], sem_sc, add=True)
                if feature_split:
                    @pl.when(c == 0)
                    def _():
                        pltpu.async_copy(ev.at[pl.ds(eb_j, 128)],
                                         den_sp.at[idx_d.at[pb + j]],
                                         sem_den, add=True)
                    den_descs.append(pltpu.make_async_copy(
                        ev.at[pl.ds(eb_j, 128)],
                        den_sp.at[idx_d.at[pb + j]], sem_den))
                if j + 2 < 16:
                    gdesc[j + 2] = _fire_rows(pb, j + 2, (j + 2) % 3)

            for d in a_descs:
                d.wait()
            _compute_e(ebn)
            sdesc[15].wait()
            if feature_split:
                @pl.when(c == 0)
                def _():
                    for d in den_descs:
                        d.wait()

        def _super_pair(tp, _):
            _one_super(tp * 2, 0)
            _one_super(tp * 2 + 1, 1)
            return 0

        lax.fori_loop(0, nsuper // 2, _super_pair, 0)
        if nsuper % 2:
            _one_super(nsuper - 1, 0)
        plsc.subcore_barrier()

        off = pl.multiple_of(s * SPSLICE, 8)

        @pl.when(c == 0)
        def _():
            pltpu.sync_copy(acc_sp.at[pl.ds(off, SPSLICE), :],
                            out_lo.at[pl.ds(off, SPSLICE), :])
            if feature_split:
                pltpu.sync_copy(den_sp.at[pl.ds(off, SPSLICE)],
                                den_o.at[pl.ds(off, SPSLICE)])

        @pl.when(c == 1)
        def _():
            pltpu.sync_copy(acc_sp.at[pl.ds(off, SPSLICE), :],
                            out_hi.at[pl.ds(off, SPSLICE), :])

    return body


def _make_sc_kernel(nv, feature_split, nsuper):
    width = 16 * nv
    out_type = (
        jax.ShapeDtypeStruct((NPAD, width), F32),
        jax.ShapeDtypeStruct((NPAD, width), F32),
        jax.ShapeDtypeStruct((NPAD,), F32),
    )
    scratch = [
        pltpu.VMEM_SHARED((SPN, width), F32),
        pltpu.VMEM_SHARED((SPN,), F32),
        pltpu.VMEM((32, 128), jnp.int32),
        pltpu.VMEM((32, 128), jnp.int32),
        pltpu.VMEM((2048,), F32),
        pltpu.VMEM((4096,), F32),
        pltpu.VMEM((128, width), F32),
        pltpu.VMEM((128, width), F32),
        pltpu.VMEM((128, width), F32),
        pltpu.VMEM((16,), F32),
        pltpu.VMEM((128,), F32),
        pltpu.SemaphoreType.DMA,
        pltpu.SemaphoreType.DMA,
        pltpu.SemaphoreType.DMA,
        pltpu.SemaphoreType.DMA,
        pltpu.SemaphoreType.DMA,
        pltpu.SemaphoreType.DMA,
        pltpu.SemaphoreType.DMA,
    ]
    return pl.kernel(
        _make_sc_body(nv, feature_split, nsuper),
        out_type=out_type,
        mesh=_MESH,
        compiler_params=_SC_PARAMS,
        scratch_types=scratch,
    )


_sc_l12 = _make_sc_kernel(2, True, SUPER)
_sc_l0 = _make_sc_kernel(1, False, SUPER0)


def kernel(pos, edge_index, W0, as0, ad0, b0, W1, as1, ad1, b1,
           W2, as2, ad2, b2):
    loop = jnp.arange(N, dtype=jnp.int32)
    src = jnp.concatenate([edge_index[0].astype(jnp.int32), loop])
    dst = jnp.concatenate([edge_index[1].astype(jnp.int32), loop])
    pad = jnp.full((EPAD - E,), N, jnp.int32)
    src2 = jnp.concatenate([src, pad]).reshape(ROWS, 128)
    dst2 = jnp.concatenate([dst, pad]).reshape(ROWS, 128)
    pos4 = jnp.concatenate([pos, jnp.ones((N, 1), F32)], axis=1)
    pos4 = jnp.pad(pos4, ((0, NPAD - N), (0, 0)))

    xa0, asrc0, adst0, m0 = _tc_a(
        pos4, W0, as0.reshape(64, 1), ad0.reshape(64, 1))
    ag0a, ag0b, _unused_den0 = _sc_l0(
        src2, dst2, asrc0.reshape(-1), adst0.reshape(-1),
        m0.reshape(-1)[:16], xa0, xa0)

    xlo1, xhi1, asrc1, adst1, m1 = _tc_b(
        ag0a, ag0b, W0, b0.reshape(1, 64), W1,
        as1.reshape(64, 1), ad1.reshape(64, 1))
    ag1lo, ag1hi, den1 = _sc_l12(
        src2, dst2, asrc1.reshape(-1), adst1.reshape(-1),
        m1.reshape(-1)[:16], xlo1, xhi1)

    xlo2, xhi2, asrc2, adst2, m2 = _tc_c(
        ag1lo, ag1hi, den1.reshape(NPAD, 1), W1, b1.reshape(1, 64), W2,
        as2.reshape(68, 1), ad2.reshape(68, 1))
    ag2lo, ag2hi, den2 = _sc_l12(
        src2, dst2, asrc2.reshape(-1), adst2.reshape(-1),
        m2.reshape(-1)[:16], xlo2, xhi2)

    (out,) = _tc_d(ag2lo, ag2hi, den2.reshape(NPAD, 1), W2,
                   b2.reshape(1, 68))
    return out[:N]
